# baseline (speedup 1.0000x reference)
import dataclasses
import functools

import jax
import jax.numpy as jnp
from jax import lax
from jax.experimental import pallas as pl
from jax.experimental.pallas import tpu as pltpu
from jax.experimental.pallas import tpu_sc as plsc

NZ, NY, NX = 10, 100, 100
NVOX = NZ * NY * NX
NVOXP = 100352
XR0, XR1 = -40.0, 40.0
YR0, YR1 = -40.0, 40.0
ZR0, ZR1 = -2.0, 6.0
VS = 0.8

N = 33600
NP = 34816
NPROW = NP // 128
SLAB = NP // 16
GRP = SLAB // 128
BW = 8
NCBLK = 33
STRIPE = NVOXP // 16
NCHUNK = NVOXP // 128
CPS = NCHUNK // 16
FSEC = 64
TILE = NY * NX

_SC_PARAMS = pltpu.CompilerParams(use_tc_tiling_on_sc=False)
if "needs_layout_passes" in pltpu.CompilerParams.__dataclass_fields__:
    _SC_PARAMS = dataclasses.replace(_SC_PARAMS, needs_layout_passes=False)


def _lerp4(v00, v01, v10, v11):
    top = v00 * 0.5 + v01 * 0.5
    bot = v10 * 0.5 + v11 * 0.5
    return top * 0.5 + bot * 0.5


def _prep_body(inp_ref, lin_ref, w_ref):
    g = inp_ref[...]
    x, y, z, cf = g[0], g[1], g[2], g[3]
    valid = jnp.isfinite(x) & jnp.isfinite(y) & jnp.isfinite(z)
    valid = valid & (cf > 1e-4)
    valid = valid & (x >= XR0) & (x < XR1)
    valid = valid & (y >= YR0) & (y < YR1)
    valid = valid & (z >= ZR0) & (z < ZR1)
    ix = jnp.clip(jnp.floor((x - XR0) / VS).astype(jnp.int32), 0, NX - 1)
    iy = jnp.clip(jnp.floor((y - YR0) / VS).astype(jnp.int32), 0, NY - 1)
    iz = jnp.clip(jnp.floor((z - ZR0) / VS).astype(jnp.int32), 0, NZ - 1)
    lin = iz * (NY * NX) + iy * NX + ix
    pid = (lax.broadcasted_iota(jnp.int32, (NPROW, 128), 0) * 128
           + lax.broadcasted_iota(jnp.int32, (NPROW, 128), 1))
    lin_ref[...] = jnp.where(valid, lin, pid & 2047)
    w_ref[...] = cf * valid.astype(jnp.float32)


def _weight_body(feat_ref, w_ref, fw1_ref, fw2_ref, warr_ref):
    w = w_ref[...]
    fw1_ref[...] = feat_ref[:, :128] * w
    fw2_ref[...] = feat_ref[:, 128:] * w
    warr_ref[...] = jnp.broadcast_to(w, w_ref.shape[:1] + (128,))


def _sc_scatter_body(fw1, fw2, warr, lin, zhbm, vol1, vol2, ws, flags_out,
                     idx_buf, upd, zbuf, flagbuf, fl16, flin, fcmp, acc,
                     fshare, sem):
    c = lax.axis_index("c")
    s = lax.axis_index("s")

    pltpu.sync_copy(zhbm, zbuf)
    z16 = jnp.zeros((16,), jnp.float32)

    @pl.loop(0, FSEC)
    def _(i):
        flagbuf[pl.ds(i * 16, 16)] = z16

    pltpu.sync_copy(lin.at[s], idx_buf)

    ones16 = jnp.ones((16,), jnp.float32)

    @pl.loop(0, GRP)
    def _(j):
        for k in range(8):
            v = idx_buf[j, pl.ds(k * 16, 16)]
            chunk = jax.lax.shift_right_logical(v, 7)
            sec = jax.lax.shift_right_logical(chunk * 2675, 17)
            slot = chunk + sec * (FSEC - CPS)
            plsc.store_scatter(flagbuf, [slot], ones16)

    pltpu.sync_copy(flagbuf, fshare.at[s])
    plsc.subcore_barrier()

    pltpu.sync_copy(fshare.at[:, pl.ds(s * FSEC, FSEC)], fl16)
    iota16 = jax.lax.iota(jnp.int32, 16)
    for k in range(FSEC // 16):
        accv = jnp.zeros((16,), jnp.float32)
        for r in range(16):
            accv = accv + fl16[r, pl.ds(k * 16, 16)]
        fcmp[pl.ds(k * 16, 16)] = accv
        if k * 16 < CPS:
            plsc.store_scatter(flin, [iota16 * 16 + k * 256], accv,
                               mask=(iota16 + k * 16) < CPS)

    @pl.when(c == 0)
    def _():
        pltpu.sync_copy(fcmp, flags_out.at[s])

    base = s * SLAB
    for cb in range(NCBLK):
        owner = 0 if cb < 17 else 1

        @pl.when(c == owner)
        def _(cb=cb):
            @pl.loop(0, CPS)
            def _(g):
                fv = jnp.max(flin[pl.ds(g * 16, 16)])

                @pl.when(fv > 0.0)
                def _():
                    pltpu.sync_copy(
                        zbuf, acc.at[pl.ds((s * CPS + g) * 128, 128), :])

            plsc.subcore_barrier()

            if cb < 16:
                src = fw1.at[pl.ds(base, SLAB), pl.ds(cb * BW, BW)]
            elif cb < 32:
                src = fw2.at[pl.ds(base, SLAB), pl.ds((cb - 16) * BW, BW)]
            else:
                src = warr.at[pl.ds(base, SLAB), pl.ds(0, BW)]
            pltpu.sync_copy(src, upd)

            @pl.loop(0, GRP)
            def _(j):
                pltpu.async_copy(upd.at[pl.ds(j * 128, 128), :],
                                 acc.at[idx_buf.at[j]], sem, add=True)

            @pl.loop(0, GRP)
            def _(j):
                pltpu.make_async_copy(upd.at[pl.ds(j * 128, 128), :],
                                      acc.at[idx_buf.at[j]], sem).wait()

            plsc.subcore_barrier()

            if cb < 16:
                dst, col = vol1, cb * BW
            elif cb < 32:
                dst, col = vol2, (cb - 16) * BW
            else:
                dst, col = ws, 0

            @pl.loop(0, CPS)
            def _(g, dst=dst, col=col):
                fv = jnp.max(flin[pl.ds(g * 16, 16)])

                @pl.when(fv > 0.0)
                def _():
                    r = (s * CPS + g) * 128
                    pltpu.sync_copy(acc.at[pl.ds(r, 128), :],
                                    dst.at[pl.ds(r, 128), pl.ds(col, BW)])


def _proj_body(v1_ref, v2_ref, ws_ref, fc_ref, pw1_ref, pw2_ref, pb_ref,
               out_ref):
    live = fc_ref[...] > 0.0
    wmax = jnp.maximum(ws_ref[:, 0:1], 1e-6)
    va = jnp.where(live, v1_ref[...] / wmax, 0.0)
    vb = jnp.where(live, v2_ref[...] / wmax, 0.0)
    dn = (((1,), (1,)), ((), ()))
    mm = jax.lax.dot_general(pw1_ref[...], va, dimension_numbers=dn,
                             preferred_element_type=jnp.float32)
    mm = mm + jax.lax.dot_general(pw2_ref[...], vb, dimension_numbers=dn,
                                  preferred_element_type=jnp.float32)
    mm = mm + pb_ref[...]
    out_ref[...] = mm.reshape(1, 1, 64, 1, NY, NX)


@jax.jit
def _lift(inp, feat_pad, proj_w, proj_b):
    lin2, w2 = pl.pallas_call(
        _prep_body,
        out_shape=[
            jax.ShapeDtypeStruct((NPROW, 128), jnp.int32),
            jax.ShapeDtypeStruct((NPROW, 128), jnp.float32),
        ],
    )(inp)
    w_col = w2.reshape(NP)[:, None]

    wchunk = NP // 16
    fw1, fw2, warr = pl.pallas_call(
        _weight_body,
        grid=(16,),
        in_specs=[
            pl.BlockSpec((wchunk, 256), lambda i: (i, 0)),
            pl.BlockSpec((wchunk, 1), lambda i: (i, 0)),
        ],
        out_specs=[
            pl.BlockSpec((wchunk, 128), lambda i: (i, 0)),
            pl.BlockSpec((wchunk, 128), lambda i: (i, 0)),
            pl.BlockSpec((wchunk, 128), lambda i: (i, 0)),
        ],
        out_shape=[
            jax.ShapeDtypeStruct((NP, 128), jnp.float32),
            jax.ShapeDtypeStruct((NP, 128), jnp.float32),
            jax.ShapeDtypeStruct((NP, 128), jnp.float32),
        ],
    )(feat_pad, w_col)

    sc_scatter = pl.kernel(
        _sc_scatter_body,
        out_type=[
            jax.ShapeDtypeStruct((NVOXP, 128), jnp.float32),
            jax.ShapeDtypeStruct((NVOXP, 128), jnp.float32),
            jax.ShapeDtypeStruct((NVOXP, 128), jnp.float32),
            jax.ShapeDtypeStruct((16, FSEC), jnp.float32),
        ],
        mesh=plsc.VectorSubcoreMesh(core_axis_name="c", subcore_axis_name="s",
                                    num_cores=2, num_subcores=16),
        compiler_params=_SC_PARAMS,
        scratch_types=[
            pltpu.VMEM((24, 128), jnp.int32),
            pltpu.VMEM((SLAB, BW), jnp.float32),
            pltpu.VMEM((128, BW), jnp.float32),
            pltpu.VMEM((16 * FSEC,), jnp.float32),
            pltpu.VMEM((16, FSEC), jnp.float32),
            pltpu.VMEM((CPS * 16,), jnp.float32),
            pltpu.VMEM((FSEC,), jnp.float32),
            pltpu.VMEM_SHARED((NVOXP, BW), jnp.float32),
            pltpu.VMEM_SHARED((16, 16 * FSEC), jnp.float32),
            pltpu.SemaphoreType.DMA,
        ],
    )
    lin3 = jnp.pad(lin2.reshape(16, GRP, 128), ((0, 0), (0, 24 - GRP), (0, 0)))
    zhbm = jnp.zeros((128, BW), jnp.float32)
    vol1, vol2, ws, fsec = sc_scatter(fw1, fw2, warr, lin3, zhbm)
    flags = fsec[:, :CPS].reshape(NCHUNK)
    fcol = jnp.repeat(flags, 128)[:, None]

    out = pl.pallas_call(
        _proj_body,
        grid=(NZ, 4),
        in_specs=[
            pl.BlockSpec((TILE, 128), lambda i, j: (i, 0)),
            pl.BlockSpec((TILE, 128), lambda i, j: (i, 0)),
            pl.BlockSpec((TILE, 128), lambda i, j: (i, 0)),
            pl.BlockSpec((TILE, 1), lambda i, j: (i, 0)),
            pl.BlockSpec((64, 128), lambda i, j: (j, 0)),
            pl.BlockSpec((64, 128), lambda i, j: (j, 0)),
            pl.BlockSpec((64, 1), lambda i, j: (j, 0)),
        ],
        out_specs=pl.BlockSpec(
            (1, 1, 64, 1, NY, NX),
            lambda i, j: (0, 0, j, i, 0, 0)),
        out_shape=jax.ShapeDtypeStruct((1, 1, 256, NZ, NY, NX), jnp.float32),
    )(vol1, vol2, ws, fcol, proj_w[:, :128], proj_w[:, 128:],
      proj_b.reshape(256, 1))
    return out


def kernel(feat_1_4, points, points_conf, proj_w, proj_b):
    b, t, v, c, h4, w4 = feat_1_4.shape
    h, w = points.shape[3], points.shape[4]
    f32 = jnp.float32

    P = points.reshape(v, h, w, 3).astype(f32)
    Cf = points_conf.reshape(v, h, w).astype(f32)

    def lerp(t11, t12, t21, t22):
        top = t11 * 0.5 + t12 * 0.5
        bot = t21 * 0.5 + t22 * 0.5
        return top * 0.5 + bot * 0.5

    pds = lerp(P[:, 1::4, 1::4, :], P[:, 1::4, 2::4, :],
               P[:, 2::4, 1::4, :], P[:, 2::4, 2::4, :])
    cds = lerp(Cf[:, 1::4, 1::4], Cf[:, 1::4, 2::4],
               Cf[:, 2::4, 1::4], Cf[:, 2::4, 2::4])
    sc = pds.transpose(0, 2, 3, 1).reshape(N, 3)
    rows = [sc[:, 0], sc[:, 1], sc[:, 2], cds.reshape(N)]
    inp = jnp.stack(rows)
    inp = jnp.pad(inp, ((0, 0), (0, NP - N)))
    inp = inp.reshape(4, NPROW, 128)

    feat_flat = (feat_1_4.reshape(v, c, h4, w4)
                 .transpose(0, 2, 3, 1).reshape(N, c).astype(f32))
    feat_pad = jnp.pad(feat_flat, ((0, NP - N), (0, 0)))

    out = _lift(inp, feat_pad, proj_w.astype(f32), proj_b.astype(f32))
    return out.astype(feat_1_4.dtype)


# speedup vs baseline: 1.0882x; 1.0882x over previous
import dataclasses
import functools

import jax
import jax.numpy as jnp
from jax import lax
from jax.experimental import pallas as pl
from jax.experimental.pallas import tpu as pltpu
from jax.experimental.pallas import tpu_sc as plsc

NZ, NY, NX = 10, 100, 100
NVOX = NZ * NY * NX
NVOXP = 100352
XR0, XR1 = -40.0, 40.0
YR0, YR1 = -40.0, 40.0
ZR0, ZR1 = -2.0, 6.0
VS = 0.8

N = 33600
NP = 34816
NPROW = NP // 128
SLAB = NP // 16
GRP = SLAB // 128
BW = 8
NCBLK = 33
STRIPE = NVOXP // 16
NCHUNK = NVOXP // 128
CPS = NCHUNK // 16
FSEC = 64
TILE = NY * NX

_SC_PARAMS = pltpu.CompilerParams(use_tc_tiling_on_sc=False)
if "needs_layout_passes" in pltpu.CompilerParams.__dataclass_fields__:
    _SC_PARAMS = dataclasses.replace(_SC_PARAMS, needs_layout_passes=False)


def _lerp4(v00, v01, v10, v11):
    top = v00 * 0.5 + v01 * 0.5
    bot = v10 * 0.5 + v11 * 0.5
    return top * 0.5 + bot * 0.5


def _prep_body(inp_ref, lin_ref, w_ref):
    g = inp_ref[...]
    x, y, z, cf = g[0], g[1], g[2], g[3]
    valid = jnp.isfinite(x) & jnp.isfinite(y) & jnp.isfinite(z)
    valid = valid & (cf > 1e-4)
    valid = valid & (x >= XR0) & (x < XR1)
    valid = valid & (y >= YR0) & (y < YR1)
    valid = valid & (z >= ZR0) & (z < ZR1)
    ix = jnp.clip(jnp.floor((x - XR0) / VS).astype(jnp.int32), 0, NX - 1)
    iy = jnp.clip(jnp.floor((y - YR0) / VS).astype(jnp.int32), 0, NY - 1)
    iz = jnp.clip(jnp.floor((z - ZR0) / VS).astype(jnp.int32), 0, NZ - 1)
    lin = iz * (NY * NX) + iy * NX + ix
    pid = (lax.broadcasted_iota(jnp.int32, (NPROW, 128), 0) * 128
           + lax.broadcasted_iota(jnp.int32, (NPROW, 128), 1))
    lin_ref[...] = jnp.where(valid, lin, pid & 2047)
    w_ref[...] = cf * valid.astype(jnp.float32)


def _weight_body(feat_ref, w_ref, fw1_ref, fw2_ref, warr_ref):
    w = w_ref[...]
    fw1_ref[...] = feat_ref[:, :128] * w
    fw2_ref[...] = feat_ref[:, 128:] * w
    warr_ref[...] = jnp.broadcast_to(w, w_ref.shape[:1] + (128,))


def _sc_scatter_body(fw1, fw2, warr, lin, zhbm, vol1, vol2, ws,
                     idx_buf, upd, zbuf, acc, sem):
    c = lax.axis_index("c")
    s = lax.axis_index("s")

    pltpu.sync_copy(zhbm, zbuf)

    pltpu.sync_copy(lin.at[s], idx_buf)

    base = s * SLAB
    for cb in range(NCBLK):
        owner = 0 if cb < 17 else 1

        @pl.when(c == owner)
        def _(cb=cb):
            pltpu.sync_copy(zbuf, acc.at[pl.ds(s * STRIPE, STRIPE), :])
            plsc.subcore_barrier()

            if cb < 16:
                src = fw1.at[pl.ds(base, SLAB), pl.ds(cb * BW, BW)]
            elif cb < 32:
                src = fw2.at[pl.ds(base, SLAB), pl.ds((cb - 16) * BW, BW)]
            else:
                src = warr.at[pl.ds(base, SLAB), pl.ds(0, BW)]
            pltpu.sync_copy(src, upd)

            @pl.loop(0, GRP)
            def _(j):
                pltpu.async_copy(upd.at[pl.ds(j * 128, 128), :],
                                 acc.at[idx_buf.at[j]], sem, add=True)

            @pl.loop(0, GRP)
            def _(j):
                pltpu.make_async_copy(upd.at[pl.ds(j * 128, 128), :],
                                      acc.at[idx_buf.at[j]], sem).wait()

            plsc.subcore_barrier()

            if cb < 16:
                dst = vol1.at[pl.ds(s * STRIPE, STRIPE), pl.ds(cb * BW, BW)]
            elif cb < 32:
                dst = vol2.at[pl.ds(s * STRIPE, STRIPE),
                              pl.ds((cb - 16) * BW, BW)]
            else:
                dst = ws.at[pl.ds(s * STRIPE, STRIPE), pl.ds(0, BW)]
            pltpu.sync_copy(acc.at[pl.ds(s * STRIPE, STRIPE), :], dst)
            plsc.subcore_barrier()


def _proj_body(v1_ref, v2_ref, ws_ref, pw1_ref, pw2_ref, pb_ref, out_ref):
    wmax = jnp.maximum(ws_ref[:, 0:1], 1e-6)
    va = v1_ref[...] / wmax
    vb = v2_ref[...] / wmax
    dn = (((1,), (1,)), ((), ()))
    mm = jax.lax.dot_general(pw1_ref[...], va, dimension_numbers=dn,
                             preferred_element_type=jnp.float32)
    mm = mm + jax.lax.dot_general(pw2_ref[...], vb, dimension_numbers=dn,
                                  preferred_element_type=jnp.float32)
    mm = mm + pb_ref[...]
    out_ref[...] = mm.reshape(1, 1, 64, 1, NY, NX)


@jax.jit
def _lift(inp, feat_pad, proj_w, proj_b):
    lin2, w2 = pl.pallas_call(
        _prep_body,
        out_shape=[
            jax.ShapeDtypeStruct((NPROW, 128), jnp.int32),
            jax.ShapeDtypeStruct((NPROW, 128), jnp.float32),
        ],
    )(inp)
    w_col = w2.reshape(NP)[:, None]

    wchunk = NP // 16
    fw1, fw2, warr = pl.pallas_call(
        _weight_body,
        grid=(16,),
        in_specs=[
            pl.BlockSpec((wchunk, 256), lambda i: (i, 0)),
            pl.BlockSpec((wchunk, 1), lambda i: (i, 0)),
        ],
        out_specs=[
            pl.BlockSpec((wchunk, 128), lambda i: (i, 0)),
            pl.BlockSpec((wchunk, 128), lambda i: (i, 0)),
            pl.BlockSpec((wchunk, 128), lambda i: (i, 0)),
        ],
        out_shape=[
            jax.ShapeDtypeStruct((NP, 128), jnp.float32),
            jax.ShapeDtypeStruct((NP, 128), jnp.float32),
            jax.ShapeDtypeStruct((NP, 128), jnp.float32),
        ],
    )(feat_pad, w_col)

    sc_scatter = pl.kernel(
        _sc_scatter_body,
        out_type=[
            jax.ShapeDtypeStruct((NVOXP, 128), jnp.float32),
            jax.ShapeDtypeStruct((NVOXP, 128), jnp.float32),
            jax.ShapeDtypeStruct((NVOXP, 128), jnp.float32),
        ],
        mesh=plsc.VectorSubcoreMesh(core_axis_name="c", subcore_axis_name="s",
                                    num_cores=2, num_subcores=16),
        compiler_params=_SC_PARAMS,
        scratch_types=[
            pltpu.VMEM((24, 128), jnp.int32),
            pltpu.VMEM((SLAB, BW), jnp.float32),
            pltpu.VMEM((STRIPE, BW), jnp.float32),
            pltpu.VMEM_SHARED((NVOXP, BW), jnp.float32),
            pltpu.SemaphoreType.DMA,
        ],
    )
    lin3 = jnp.pad(lin2.reshape(16, GRP, 128), ((0, 0), (0, 24 - GRP), (0, 0)))
    zhbm = jnp.zeros((STRIPE, BW), jnp.float32)
    vol1, vol2, ws = sc_scatter(fw1, fw2, warr, lin3, zhbm)

    out = pl.pallas_call(
        _proj_body,
        grid=(NZ, 4),
        in_specs=[
            pl.BlockSpec((TILE, 128), lambda i, j: (i, 0)),
            pl.BlockSpec((TILE, 128), lambda i, j: (i, 0)),
            pl.BlockSpec((TILE, 128), lambda i, j: (i, 0)),
            pl.BlockSpec((64, 128), lambda i, j: (j, 0)),
            pl.BlockSpec((64, 128), lambda i, j: (j, 0)),
            pl.BlockSpec((64, 1), lambda i, j: (j, 0)),
        ],
        out_specs=pl.BlockSpec(
            (1, 1, 64, 1, NY, NX),
            lambda i, j: (0, 0, j, i, 0, 0)),
        out_shape=jax.ShapeDtypeStruct((1, 1, 256, NZ, NY, NX), jnp.float32),
    )(vol1, vol2, ws, proj_w[:, :128], proj_w[:, 128:],
      proj_b.reshape(256, 1))
    return out


def kernel(feat_1_4, points, points_conf, proj_w, proj_b):
    b, t, v, c, h4, w4 = feat_1_4.shape
    h, w = points.shape[3], points.shape[4]
    f32 = jnp.float32

    P = points.reshape(v, h, w, 3).astype(f32)
    Cf = points_conf.reshape(v, h, w).astype(f32)

    def lerp(t11, t12, t21, t22):
        top = t11 * 0.5 + t12 * 0.5
        bot = t21 * 0.5 + t22 * 0.5
        return top * 0.5 + bot * 0.5

    pds = lerp(P[:, 1::4, 1::4, :], P[:, 1::4, 2::4, :],
               P[:, 2::4, 1::4, :], P[:, 2::4, 2::4, :])
    cds = lerp(Cf[:, 1::4, 1::4], Cf[:, 1::4, 2::4],
               Cf[:, 2::4, 1::4], Cf[:, 2::4, 2::4])
    sc = pds.transpose(0, 2, 3, 1).reshape(N, 3)
    rows = [sc[:, 0], sc[:, 1], sc[:, 2], cds.reshape(N)]
    inp = jnp.stack(rows)
    inp = jnp.pad(inp, ((0, 0), (0, NP - N)))
    inp = inp.reshape(4, NPROW, 128)

    feat_flat = (feat_1_4.reshape(v, c, h4, w4)
                 .transpose(0, 2, 3, 1).reshape(N, c).astype(f32))
    feat_pad = jnp.pad(feat_flat, ((0, NP - N), (0, 0)))

    out = _lift(inp, feat_pad, proj_w.astype(f32), proj_b.astype(f32))
    return out.astype(feat_1_4.dtype)
